# Initial kernel scaffold; baseline (speedup 1.0000x reference)
import jax
import jax.numpy as jnp
from jax.experimental import pallas as pl


def kernel(x, angles, gate, bias, plane_i, plane_j):
    raise NotImplementedError("write your pallas kernel here")


# speedup vs baseline: 4.3965x; 4.3965x over previous
import functools

import jax
import jax.numpy as jnp
from jax.experimental import pallas as pl
from jax.experimental.pallas import tpu as pltpu

HIDDEN = 1024
N_PASSES = 3
BLOCK_ROWS = 2048


def _rot_matrices(angles, plane_i, plane_j):
    cos_a = jnp.cos(angles)
    sin_a = jnp.sin(angles)
    p = jnp.arange(N_PASSES)[:, None]
    diag = jnp.ones((N_PASSES, HIDDEN), jnp.float32)
    diag = diag.at[p, plane_i].set(cos_a).at[p, plane_j].set(cos_a)
    R = jnp.zeros((N_PASSES, HIDDEN, HIDDEN), jnp.float32)
    R = R.at[p, plane_j, plane_i].set(-sin_a)
    R = R.at[p, plane_i, plane_j].set(sin_a)
    d = jnp.arange(HIDDEN)[None, :]
    R = R.at[p, d, d].set(diag)
    return R


def _body(x_ref, R_ref, gate_ref, bias_ref, o_ref):
    h = x_ref[...]
    for pp in range(N_PASSES):
        h = jnp.dot(h, R_ref[pp], preferred_element_type=jnp.float32)
        z = h * gate_ref[pp][None, :] + bias_ref[pp][None, :]
        h = z / (1.0 + jnp.exp(-z))
    o_ref[...] = h


def kernel(x, angles, gate, bias, plane_i, plane_j):
    orig_shape = x.shape
    h = x.reshape(-1, HIDDEN)
    n_rows = h.shape[0]
    R = _rot_matrices(angles, plane_i, plane_j)
    grid = (n_rows // BLOCK_ROWS,)
    out = pl.pallas_call(
        _body,
        grid=grid,
        in_specs=[
            pl.BlockSpec((BLOCK_ROWS, HIDDEN), lambda i: (i, 0)),
            pl.BlockSpec((N_PASSES, HIDDEN, HIDDEN), lambda i: (0, 0, 0)),
            pl.BlockSpec((N_PASSES, HIDDEN), lambda i: (0, 0)),
            pl.BlockSpec((N_PASSES, HIDDEN), lambda i: (0, 0)),
        ],
        out_specs=pl.BlockSpec((BLOCK_ROWS, HIDDEN), lambda i: (i, 0)),
        out_shape=jax.ShapeDtypeStruct((n_rows, HIDDEN), jnp.float32),
    )(h, R, gate, bias)
    return out.reshape(orig_shape)
